# Initial kernel scaffold; baseline (speedup 1.0000x reference)
import jax
import jax.numpy as jnp
from jax.experimental import pallas as pl


def kernel(tokens, emb, pos, init_state, init_val, Ws, Wd, rw, Ps, Pt, pw, ln_g, ln_b):
    raise NotImplementedError("write your pallas kernel here")


# speedup vs baseline: 1.1778x; 1.1778x over previous
import math

import jax
import jax.numpy as jnp
from jax import lax
from jax.experimental import pallas as pl
from jax.experimental.pallas import tpu as pltpu

K = 16
P = 64
STATE_MASS = 4.0
VB = 2048


def _mm_default(a, bm, dims):
    return lax.dot_general(a.astype(jnp.bfloat16), bm.astype(jnp.bfloat16),
                           dims, preferred_element_type=jnp.float32)


def _mm_exact(a, bm, dims):
    return lax.dot_general(a, bm, dims, precision=lax.Precision.HIGHEST,
                           preferred_element_type=jnp.float32)


_NT = (((1,), (1,)), ((), ()))
_NN = (((1,), (0,)), ((), ()))
_TN = (((0,), (0,)), ((), ()))


def _ln(x, g, b):
    mu = x.mean(-1, keepdims=True)
    var = ((x - mu) ** 2).mean(-1, keepdims=True)
    return (x - mu) / jnp.sqrt(var + 1e-5) * g + b


def _signed_softmax_state(s):
    s = jnp.nan_to_num(s)
    a = jnp.abs(s)
    m = jnp.max(a, axis=-1, keepdims=True)
    e = jnp.exp(a - m)
    return jnp.sign(s) * e / jnp.sum(e, axis=-1, keepdims=True) * STATE_MASS


def _topk_signed_softmax_dense(s):
    s = jnp.nan_to_num(s)
    work = s
    t = None
    for _ in range(K):
        t = jnp.max(work, axis=-1, keepdims=True)
        work = jnp.where(work == t, -1e30, work)
    mask = s >= t
    a = jnp.abs(s)
    m1 = jnp.max(jnp.where(mask, a, -1e30), axis=-1, keepdims=True)
    e = jnp.where(mask, jnp.exp(a - m1), 0.0)
    return jnp.sign(s) * e / jnp.sum(e, axis=-1, keepdims=True)


def _levels_kernel(x_ref, istate_ref, ival_ref, ws_ref, wd_ref, rw_ref,
                   ps_ref, pt_ref, pw_ref, g_ref, b_ref, h_ref):
    S = x_ref.shape[1]
    L, M = istate_ref.shape
    D = x_ref.shape[2]
    src = x_ref[0]
    mem_vals = []
    mem_states = []
    for l in range(L):
        g = g_ref[l][None, :]
        bb = b_ref[l][None, :]
        state0 = _signed_softmax_state(istate_ref[l][None, :])
        val0 = _ln(ival_ref[l], g, bb)
        qs = _mm_default(src, ws_ref[l], _NN) * rw_ref[l][None, :]
        ksrc = _mm_default(val0, wd_ref[l], _NN)
        scores = _mm_default(qs, ksrc, _NT)
        w_full = _topk_signed_softmax_dense(scores)
        dval = _mm_exact(w_full, src, _TN)
        dstate = jnp.sum(w_full, axis=0, keepdims=True)
        state = _signed_softmax_state(state0 + dstate)
        val = _ln(val0 + dval, g, bb)
        pq = _mm_default(val, ps_ref[l], _NN) * pw_ref[l][None, :]
        pk = _mm_default(val, pt_ref[l], _NN)
        pscores = _mm_default(pq, pk, _NT) * state
        we_full = _topk_signed_softmax_dense(pscores)
        agg = _mm_exact(we_full, val, _NN)
        val = _ln(val + agg, g, bb)
        mem_vals.append(val)
        mem_states.append(state)
        src = val * jnp.reshape(state, (M, 1))
    memcat = jnp.concatenate(mem_vals, axis=0)
    statecat = jnp.concatenate(mem_states, axis=1)
    q = x_ref[0, S - P:, :]
    att_s = _mm_default(q, memcat, _NT)
    att_s = att_s / math.sqrt(D) + statecat
    att_m = jnp.max(att_s, axis=-1, keepdims=True)
    e = jnp.exp(att_s - att_m)
    att = e / jnp.sum(e, axis=-1, keepdims=True)
    read = _mm_default(att, memcat, _NN)
    h_ref[0] = q + read


def _gather_kernel(tok_ref, emb_ref, pos_ref, x_ref):
    x_ref[...] = emb_ref[...] + pos_ref[...]


def _logits_kernel(h_ref, emb_ref, o_ref):
    o_ref[...] = _mm_default(h_ref[...], emb_ref[...], _NT)


def kernel(tokens, emb, pos, init_state, init_val, Ws, Wd, rw, Ps, Pt, pw, ln_g, ln_b):
    B, S = tokens.shape
    V, D = emb.shape
    L, M = init_state.shape

    grid_spec = pltpu.PrefetchScalarGridSpec(
        num_scalar_prefetch=1,
        grid=(B * S,),
        in_specs=[
            pl.BlockSpec((1, 1, D), lambda i, tok: (tok[i], 0, 0)),
            pl.BlockSpec((1, 1, D), lambda i, tok: (i % S, 0, 0)),
        ],
        out_specs=pl.BlockSpec((1, 1, D), lambda i, tok: (i, 0, 0)),
    )
    x = pl.pallas_call(
        _gather_kernel,
        grid_spec=grid_spec,
        out_shape=jax.ShapeDtypeStruct((B * S, 1, D), jnp.float32),
    )(tokens.reshape(-1), emb.reshape(V, 1, D), pos.reshape(S, 1, D))
    x = x.reshape(B, S, D)

    full = lambda a: pl.BlockSpec(a.shape, lambda bidx: (0,) * a.ndim)
    h = pl.pallas_call(
        _levels_kernel,
        grid=(B,),
        in_specs=[
            pl.BlockSpec((1, S, D), lambda bidx: (bidx, 0, 0)),
            full(init_state), full(init_val), full(Ws), full(Wd), full(rw),
            full(Ps), full(Pt), full(pw), full(ln_g), full(ln_b),
        ],
        out_specs=pl.BlockSpec((1, P, D), lambda bidx: (bidx, 0, 0)),
        out_shape=jax.ShapeDtypeStruct((B, P, D), jnp.float32),
    )(x, init_state, init_val, Ws, Wd, rw, Ps, Pt, pw, ln_g, ln_b)

    h2 = h.reshape(B * P, D)
    logits = pl.pallas_call(
        _logits_kernel,
        grid=(pl.cdiv(V, VB),),
        in_specs=[
            pl.BlockSpec((B * P, D), lambda i: (0, 0)),
            pl.BlockSpec((VB, D), lambda i: (i, 0)),
        ],
        out_specs=pl.BlockSpec((B * P, VB), lambda i: (0, i)),
        out_shape=jax.ShapeDtypeStruct((B * P, V), jnp.float32),
    )(h2, emb)
    return logits.reshape(B, P, V)


# speedup vs baseline: 10.3298x; 8.7706x over previous
import functools
import math

import jax
import jax.numpy as jnp
from jax import lax
from jax.experimental import pallas as pl
from jax.experimental.pallas import tpu as pltpu
from jax.experimental.pallas import tpu_sc as plsc

K = 16
P = 64
STATE_MASS = 4.0
VB = 2048


def _mm_default(a, bm, dims):
    return lax.dot_general(a.astype(jnp.bfloat16), bm.astype(jnp.bfloat16),
                           dims, preferred_element_type=jnp.float32)


def _mm_exact(a, bm, dims):
    return lax.dot_general(a, bm, dims, precision=lax.Precision.HIGHEST,
                           preferred_element_type=jnp.float32)


_NT = (((1,), (1,)), ((), ()))
_NN = (((1,), (0,)), ((), ()))
_TN = (((0,), (0,)), ((), ()))


def _ln(x, g, b):
    mu = x.mean(-1, keepdims=True)
    var = ((x - mu) ** 2).mean(-1, keepdims=True)
    return (x - mu) / jnp.sqrt(var + 1e-5) * g + b


def _signed_softmax_state(s):
    s = jnp.nan_to_num(s)
    a = jnp.abs(s)
    m = jnp.max(a, axis=-1, keepdims=True)
    e = jnp.exp(a - m)
    return jnp.sign(s) * e / jnp.sum(e, axis=-1, keepdims=True) * STATE_MASS


def _topk_signed_softmax_dense(s):
    s = jnp.nan_to_num(s)
    work = s
    t = None
    for _ in range(K):
        t = jnp.max(work, axis=-1, keepdims=True)
        work = jnp.where(work == t, -1e30, work)
    mask = s >= t
    a = jnp.abs(s)
    m1 = jnp.max(jnp.where(mask, a, -1e30), axis=-1, keepdims=True)
    e = jnp.where(mask, jnp.exp(a - m1), 0.0)
    return jnp.sign(s) * e / jnp.sum(e, axis=-1, keepdims=True)


def _levels_kernel(x_ref, pos_ref, istate_ref, ival_ref, ws_ref, wd_ref, rw_ref,
                   ps_ref, pt_ref, pw_ref, g_ref, b_ref, h_ref):
    S = x_ref.shape[1]
    L, M = istate_ref.shape
    D = x_ref.shape[2]
    src = x_ref[0] + pos_ref[...]
    mem_vals = []
    mem_states = []
    for l in range(L):
        g = g_ref[l][None, :]
        bb = b_ref[l][None, :]
        state0 = _signed_softmax_state(istate_ref[l][None, :])
        val0 = _ln(ival_ref[l], g, bb)
        qs = _mm_default(src, ws_ref[l], _NN) * rw_ref[l][None, :]
        ksrc = _mm_default(val0, wd_ref[l], _NN)
        scores = _mm_default(qs, ksrc, _NT)
        w_full = _topk_signed_softmax_dense(scores)
        dval = _mm_exact(w_full, src, _TN)
        dstate = jnp.sum(w_full, axis=0, keepdims=True)
        state = _signed_softmax_state(state0 + dstate)
        val = _ln(val0 + dval, g, bb)
        pq = _mm_default(val, ps_ref[l], _NN) * pw_ref[l][None, :]
        pk = _mm_default(val, pt_ref[l], _NN)
        pscores = _mm_default(pq, pk, _NT) * state
        we_full = _topk_signed_softmax_dense(pscores)
        agg = _mm_exact(we_full, val, _NN)
        val = _ln(val + agg, g, bb)
        mem_vals.append(val)
        mem_states.append(state)
        src = val * jnp.reshape(state, (M, 1))
    memcat = jnp.concatenate(mem_vals, axis=0)
    statecat = jnp.concatenate(mem_states, axis=1)
    q = x_ref[0, S - P:, :] + pos_ref[S - P:, :]
    att_s = _mm_default(q, memcat, _NT)
    att_s = att_s / math.sqrt(D) + statecat
    att_m = jnp.max(att_s, axis=-1, keepdims=True)
    e = jnp.exp(att_s - att_m)
    att = e / jnp.sum(e, axis=-1, keepdims=True)
    read = _mm_default(att, memcat, _NN)
    h_ref[0] = q + read


def _sc_gather(emb, tok_flat):
    V, D = emb.shape
    n_tok = tok_flat.shape[0]
    info = plsc.get_sparse_core_info()
    nw = info.num_cores * info.num_subcores
    b_per_w = n_tok // nw
    mesh = plsc.VectorSubcoreMesh(core_axis_name="c", subcore_axis_name="s")

    @functools.partial(
        pl.kernel, mesh=mesh,
        out_type=jax.ShapeDtypeStruct((n_tok, D), jnp.float32),
        scratch_types=[
            pltpu.VMEM((b_per_w,), jnp.int32),
            pltpu.VMEM((b_per_w, D), jnp.float32),
            pltpu.SemaphoreType.DMA,
        ],
    )
    def k(emb_hbm, idx_hbm, out_hbm, idx_v, rows_v, sem):
        wid = lax.axis_index("s") * info.num_cores + lax.axis_index("c")
        base = wid * b_per_w
        pltpu.sync_copy(idx_hbm.at[pl.ds(base, b_per_w)], idx_v)
        pltpu.async_copy(emb_hbm.at[idx_v], rows_v, sem).wait()
        pltpu.sync_copy(rows_v, out_hbm.at[pl.ds(base, b_per_w)])

    return k(emb, tok_flat)


def _logits_kernel(h_ref, emb_ref, o_ref):
    o_ref[...] = _mm_default(h_ref[...], emb_ref[...], _NT)


def kernel(tokens, emb, pos, init_state, init_val, Ws, Wd, rw, Ps, Pt, pw, ln_g, ln_b):
    B, S = tokens.shape
    V, D = emb.shape
    L, M = init_state.shape

    x = _sc_gather(emb, tokens.reshape(-1).astype(jnp.int32))
    x = x.reshape(B, S, D)

    full = lambda a: pl.BlockSpec(a.shape, lambda bidx: (0,) * a.ndim)
    h = pl.pallas_call(
        _levels_kernel,
        grid=(B,),
        in_specs=[
            pl.BlockSpec((1, S, D), lambda bidx: (bidx, 0, 0)),
            full(pos),
            full(init_state), full(init_val), full(Ws), full(Wd), full(rw),
            full(Ps), full(Pt), full(pw), full(ln_g), full(ln_b),
        ],
        out_specs=pl.BlockSpec((1, P, D), lambda bidx: (bidx, 0, 0)),
        out_shape=jax.ShapeDtypeStruct((B, P, D), jnp.float32),
    )(x, pos, init_state, init_val, Ws, Wd, rw, Ps, Pt, pw, ln_g, ln_b)

    h2 = h.reshape(B * P, D)
    logits = pl.pallas_call(
        _logits_kernel,
        grid=(pl.cdiv(V, VB),),
        in_specs=[
            pl.BlockSpec((B * P, D), lambda i: (0, 0)),
            pl.BlockSpec((VB, D), lambda i: (i, 0)),
        ],
        out_specs=pl.BlockSpec((B * P, VB), lambda i: (0, i)),
        out_shape=jax.ShapeDtypeStruct((B * P, V), jnp.float32),
    )(h2, emb)
    return logits.reshape(B, P, V)


# speedup vs baseline: 10.3936x; 1.0062x over previous
import functools
import math

import jax
import jax.numpy as jnp
from jax import lax
from jax.experimental import pallas as pl
from jax.experimental.pallas import tpu as pltpu
from jax.experimental.pallas import tpu_sc as plsc

K = 16
P = 64
STATE_MASS = 4.0
VB = 2048


def _mm_default(a, bm, dims):
    return lax.dot_general(a.astype(jnp.bfloat16), bm.astype(jnp.bfloat16),
                           dims, preferred_element_type=jnp.float32)


def _mm_exact(a, bm, dims):
    return lax.dot_general(a, bm, dims, precision=lax.Precision.HIGHEST,
                           preferred_element_type=jnp.float32)


_NT = (((1,), (1,)), ((), ()))
_NN = (((1,), (0,)), ((), ()))
_TN = (((0,), (0,)), ((), ()))


def _ln(x, g, b):
    mu = x.mean(-1, keepdims=True)
    var = ((x - mu) ** 2).mean(-1, keepdims=True)
    return (x - mu) / jnp.sqrt(var + 1e-5) * g + b


def _signed_softmax_state(s):
    s = jnp.nan_to_num(s)
    a = jnp.abs(s)
    m = jnp.max(a, axis=-1, keepdims=True)
    e = jnp.exp(a - m)
    return jnp.sign(s) * e / jnp.sum(e, axis=-1, keepdims=True) * STATE_MASS


def _topk_signed_softmax_dense(s):
    s = jnp.nan_to_num(s)
    n, mw = s.shape
    c0, c1, c2, c3 = (s[:, j * 128:(j + 1) * 128] for j in range(4))
    lo01, hi01 = jnp.minimum(c0, c1), jnp.maximum(c0, c1)
    lo23, hi23 = jnp.minimum(c2, c3), jnp.maximum(c2, c3)
    r1, a02 = jnp.maximum(hi01, hi23), jnp.minimum(hi01, hi23)
    b13, r4 = jnp.maximum(lo01, lo23), jnp.minimum(lo01, lo23)
    r2, r3 = jnp.maximum(a02, b13), jnp.minimum(a02, b13)
    t = None
    for _ in range(K):
        t = jnp.max(r1, axis=-1, keepdims=True)
        pop = r1 == t
        r1 = jnp.where(pop, r2, r1)
        r2 = jnp.where(pop, r3, r2)
        r3 = jnp.where(pop, r4, r3)
        r4 = jnp.where(pop, -1e30, r4)
    mask = s >= t
    a = jnp.abs(s)
    m1 = jnp.max(jnp.where(mask, a, -1e30), axis=-1, keepdims=True)
    e = jnp.where(mask, jnp.exp(a - m1), 0.0)
    return jnp.sign(s) * e / jnp.sum(e, axis=-1, keepdims=True)


def _levels_kernel(x_ref, pos_ref, istate_ref, ival_ref, ws_ref, wd_ref, rw_ref,
                   ps_ref, pt_ref, pw_ref, g_ref, b_ref, h_ref):
    S = x_ref.shape[1]
    L, M = istate_ref.shape
    D = x_ref.shape[2]
    src = x_ref[0] + pos_ref[...]
    mem_vals = []
    mem_states = []
    for l in range(L):
        g = g_ref[l][None, :]
        bb = b_ref[l][None, :]
        state0 = _signed_softmax_state(istate_ref[l][None, :])
        val0 = _ln(ival_ref[l], g, bb)
        qs = _mm_default(src, ws_ref[l], _NN) * rw_ref[l][None, :]
        ksrc = _mm_default(val0, wd_ref[l], _NN)
        scores = _mm_default(qs, ksrc, _NT)
        w_full = _topk_signed_softmax_dense(scores)
        dval = _mm_exact(w_full, src, _TN)
        dstate = jnp.sum(w_full, axis=0, keepdims=True)
        state = _signed_softmax_state(state0 + dstate)
        val = _ln(val0 + dval, g, bb)
        pq = _mm_default(val, ps_ref[l], _NN) * pw_ref[l][None, :]
        pk = _mm_default(val, pt_ref[l], _NN)
        pscores = _mm_default(pq, pk, _NT) * state
        we_full = _topk_signed_softmax_dense(pscores)
        agg = _mm_exact(we_full, val, _NN)
        val = _ln(val + agg, g, bb)
        mem_vals.append(val)
        mem_states.append(state)
        src = val * jnp.reshape(state, (M, 1))
    memcat = jnp.concatenate(mem_vals, axis=0)
    statecat = jnp.concatenate(mem_states, axis=1)
    q = x_ref[0, S - P:, :] + pos_ref[S - P:, :]
    att_s = _mm_default(q, memcat, _NT)
    att_s = att_s / math.sqrt(D) + statecat
    att_m = jnp.max(att_s, axis=-1, keepdims=True)
    e = jnp.exp(att_s - att_m)
    att = e / jnp.sum(e, axis=-1, keepdims=True)
    read = _mm_default(att, memcat, _NN)
    h_ref[0] = q + read


def _sc_gather(emb, tok_flat):
    V, D = emb.shape
    n_tok = tok_flat.shape[0]
    info = plsc.get_sparse_core_info()
    nw = info.num_cores * info.num_subcores
    b_per_w = n_tok // nw
    mesh = plsc.VectorSubcoreMesh(core_axis_name="c", subcore_axis_name="s")

    @functools.partial(
        pl.kernel, mesh=mesh,
        out_type=jax.ShapeDtypeStruct((n_tok, D), jnp.float32),
        scratch_types=[
            pltpu.VMEM((b_per_w,), jnp.int32),
            pltpu.VMEM((b_per_w, D), jnp.float32),
            pltpu.SemaphoreType.DMA,
        ],
    )
    def k(emb_hbm, idx_hbm, out_hbm, idx_v, rows_v, sem):
        wid = lax.axis_index("s") * info.num_cores + lax.axis_index("c")
        base = wid * b_per_w
        pltpu.sync_copy(idx_hbm.at[pl.ds(base, b_per_w)], idx_v)
        pltpu.async_copy(emb_hbm.at[idx_v], rows_v, sem).wait()
        pltpu.sync_copy(rows_v, out_hbm.at[pl.ds(base, b_per_w)])

    return k(emb, tok_flat)


def _logits_kernel(h_ref, emb_ref, o_ref):
    o_ref[...] = _mm_default(h_ref[...], emb_ref[...], _NT)


def kernel(tokens, emb, pos, init_state, init_val, Ws, Wd, rw, Ps, Pt, pw, ln_g, ln_b):
    B, S = tokens.shape
    V, D = emb.shape
    L, M = init_state.shape

    x = _sc_gather(emb, tokens.reshape(-1).astype(jnp.int32))
    x = x.reshape(B, S, D)

    full = lambda a: pl.BlockSpec(a.shape, lambda bidx: (0,) * a.ndim)
    h = pl.pallas_call(
        _levels_kernel,
        grid=(B,),
        in_specs=[
            pl.BlockSpec((1, S, D), lambda bidx: (bidx, 0, 0)),
            full(pos),
            full(init_state), full(init_val), full(Ws), full(Wd), full(rw),
            full(Ps), full(Pt), full(pw), full(ln_g), full(ln_b),
        ],
        out_specs=pl.BlockSpec((1, P, D), lambda bidx: (bidx, 0, 0)),
        out_shape=jax.ShapeDtypeStruct((B, P, D), jnp.float32),
    )(x, pos, init_state, init_val, Ws, Wd, rw, Ps, Pt, pw, ln_g, ln_b)

    h2 = h.reshape(B * P, D)
    logits = pl.pallas_call(
        _logits_kernel,
        grid=(pl.cdiv(V, VB),),
        in_specs=[
            pl.BlockSpec((B * P, D), lambda i: (0, 0)),
            pl.BlockSpec((VB, D), lambda i: (i, 0)),
        ],
        out_specs=pl.BlockSpec((B * P, VB), lambda i: (0, i)),
        out_shape=jax.ShapeDtypeStruct((B * P, V), jnp.float32),
    )(h2, emb)
    return logits.reshape(B, P, V)


# speedup vs baseline: 10.8825x; 1.0470x over previous
import functools
import math

import jax
import jax.numpy as jnp
from jax import lax
from jax.experimental import pallas as pl
from jax.experimental.pallas import tpu as pltpu
from jax.experimental.pallas import tpu_sc as plsc

K = 16
P = 64
STATE_MASS = 4.0
VB = 8192


def _mm_default(a, bm, dims):
    return lax.dot_general(a.astype(jnp.bfloat16), bm.astype(jnp.bfloat16),
                           dims, preferred_element_type=jnp.float32)


def _mm_exact(a, bm, dims):
    return lax.dot_general(a, bm, dims, precision=lax.Precision.HIGHEST,
                           preferred_element_type=jnp.float32)


_NT = (((1,), (1,)), ((), ()))
_NN = (((1,), (0,)), ((), ()))
_TN = (((0,), (0,)), ((), ()))


def _ln(x, g, b):
    mu = x.mean(-1, keepdims=True)
    var = ((x - mu) ** 2).mean(-1, keepdims=True)
    return (x - mu) / jnp.sqrt(var + 1e-5) * g + b


def _signed_softmax_state(s):
    s = jnp.nan_to_num(s)
    a = jnp.abs(s)
    m = jnp.max(a, axis=-1, keepdims=True)
    e = jnp.exp(a - m)
    return jnp.sign(s) * e / jnp.sum(e, axis=-1, keepdims=True) * STATE_MASS


def _topk_signed_softmax_dense(s):
    s = jnp.nan_to_num(s)
    n, mw = s.shape
    c0, c1, c2, c3 = (s[:, j * 128:(j + 1) * 128] for j in range(4))
    lo01, hi01 = jnp.minimum(c0, c1), jnp.maximum(c0, c1)
    lo23, hi23 = jnp.minimum(c2, c3), jnp.maximum(c2, c3)
    r1, a02 = jnp.maximum(hi01, hi23), jnp.minimum(hi01, hi23)
    b13, r4 = jnp.maximum(lo01, lo23), jnp.minimum(lo01, lo23)
    r2, r3 = jnp.maximum(a02, b13), jnp.minimum(a02, b13)
    t = t1 = None
    for i in range(K):
        t = jnp.max(r1, axis=-1, keepdims=True)
        if i == 0:
            t1 = t
        pop = r1 == t
        r1 = jnp.where(pop, r2, r1)
        r2 = jnp.where(pop, r3, r2)
        r3 = jnp.where(pop, r4, r3)
        r4 = jnp.where(pop, -1e30, r4)
    mask = s >= t
    m1 = jnp.maximum(t1, -t)
    e = jnp.where(mask, jnp.exp(jnp.abs(s) - m1), 0.0)
    return jnp.sign(s) * e / jnp.sum(e, axis=-1, keepdims=True)


def _levels_kernel(x_ref, pos_ref, istate_ref, ival_ref, ws_ref, wd_ref, rw_ref,
                   ps_ref, pt_ref, pw_ref, g_ref, b_ref, h_ref):
    S = x_ref.shape[1]
    L, M = istate_ref.shape
    D = x_ref.shape[2]
    src = x_ref[0] + pos_ref[...]
    mem_vals = []
    mem_states = []
    for l in range(L):
        g = g_ref[l][None, :]
        bb = b_ref[l][None, :]
        state0 = _signed_softmax_state(istate_ref[l][None, :])
        val0 = _ln(ival_ref[l], g, bb)
        qs = _mm_default(src, ws_ref[l], _NN) * rw_ref[l][None, :]
        ksrc = _mm_default(val0, wd_ref[l], _NN)
        scores = _mm_default(qs, ksrc, _NT)
        w_full = _topk_signed_softmax_dense(scores)
        dval = _mm_exact(w_full, src, _TN)
        dstate = jnp.sum(w_full, axis=0, keepdims=True)
        state = _signed_softmax_state(state0 + dstate)
        val = _ln(val0 + dval, g, bb)
        pq = _mm_default(val, ps_ref[l], _NN) * pw_ref[l][None, :]
        pk = _mm_default(val, pt_ref[l], _NN)
        pscores = _mm_default(pq, pk, _NT) * state
        we_full = _topk_signed_softmax_dense(pscores)
        agg = _mm_exact(we_full, val, _NN)
        val = _ln(val + agg, g, bb)
        mem_vals.append(val)
        mem_states.append(state)
        src = val * jnp.reshape(state, (M, 1))
    memcat = jnp.concatenate(mem_vals, axis=0)
    statecat = jnp.concatenate(mem_states, axis=1)
    q = x_ref[0, S - P:, :] + pos_ref[S - P:, :]
    att_s = _mm_default(q, memcat, _NT)
    att_s = att_s / math.sqrt(D) + statecat
    att_m = jnp.max(att_s, axis=-1, keepdims=True)
    e = jnp.exp(att_s - att_m)
    att = e / jnp.sum(e, axis=-1, keepdims=True)
    read = _mm_default(att, memcat, _NN)
    h_ref[0] = q + read


def _sc_gather(emb, tok_flat):
    V, D = emb.shape
    n_tok = tok_flat.shape[0]
    info = plsc.get_sparse_core_info()
    nw = info.num_cores * info.num_subcores
    b_per_w = n_tok // nw
    mesh = plsc.VectorSubcoreMesh(core_axis_name="c", subcore_axis_name="s")

    @functools.partial(
        pl.kernel, mesh=mesh,
        out_type=jax.ShapeDtypeStruct((n_tok, D), jnp.float32),
        scratch_types=[
            pltpu.VMEM((b_per_w,), jnp.int32),
            pltpu.VMEM((b_per_w, D), jnp.float32),
            pltpu.SemaphoreType.DMA,
        ],
    )
    def k(emb_hbm, idx_hbm, out_hbm, idx_v, rows_v, sem):
        wid = lax.axis_index("s") * info.num_cores + lax.axis_index("c")
        base = wid * b_per_w
        pltpu.sync_copy(idx_hbm.at[pl.ds(base, b_per_w)], idx_v)
        pltpu.async_copy(emb_hbm.at[idx_v], rows_v, sem).wait()
        pltpu.sync_copy(rows_v, out_hbm.at[pl.ds(base, b_per_w)])

    return k(emb, tok_flat)


def _logits_kernel(h_ref, emb_ref, o_ref):
    o_ref[...] = _mm_default(h_ref[...], emb_ref[...], _NT)


def kernel(tokens, emb, pos, init_state, init_val, Ws, Wd, rw, Ps, Pt, pw, ln_g, ln_b):
    B, S = tokens.shape
    V, D = emb.shape
    L, M = init_state.shape

    x = _sc_gather(emb, tokens.reshape(-1).astype(jnp.int32))
    x = x.reshape(B, S, D)

    full = lambda a: pl.BlockSpec(a.shape, lambda bidx: (0,) * a.ndim)
    h = pl.pallas_call(
        _levels_kernel,
        grid=(B,),
        in_specs=[
            pl.BlockSpec((1, S, D), lambda bidx: (bidx, 0, 0)),
            full(pos),
            full(init_state), full(init_val), full(Ws), full(Wd), full(rw),
            full(Ps), full(Pt), full(pw), full(ln_g), full(ln_b),
        ],
        out_specs=pl.BlockSpec((1, P, D), lambda bidx: (bidx, 0, 0)),
        out_shape=jax.ShapeDtypeStruct((B, P, D), jnp.float32),
    )(x, pos, init_state, init_val, Ws, Wd, rw, Ps, Pt, pw, ln_g, ln_b)

    h2 = h.reshape(B * P, D)
    logits = pl.pallas_call(
        _logits_kernel,
        grid=(pl.cdiv(V, VB),),
        in_specs=[
            pl.BlockSpec((B * P, D), lambda i: (0, 0)),
            pl.BlockSpec((VB, D), lambda i: (i, 0)),
        ],
        out_specs=pl.BlockSpec((B * P, VB), lambda i: (0, i)),
        out_shape=jax.ShapeDtypeStruct((B * P, V), jnp.float32),
    )(h2, emb)
    return logits.reshape(B, P, V)


# speedup vs baseline: 10.9445x; 1.0057x over previous
import functools
import math

import jax
import jax.numpy as jnp
from jax import lax
from jax.experimental import pallas as pl
from jax.experimental.pallas import tpu as pltpu
from jax.experimental.pallas import tpu_sc as plsc

K = 16
P = 64
STATE_MASS = 4.0
VB = 8192


def _mm_default(a, bm, dims):
    return lax.dot_general(a.astype(jnp.bfloat16), bm.astype(jnp.bfloat16),
                           dims, preferred_element_type=jnp.float32)


def _mm_exact(a, bm, dims):
    return lax.dot_general(a, bm, dims, precision=lax.Precision.HIGHEST,
                           preferred_element_type=jnp.float32)


_NT = (((1,), (1,)), ((), ()))
_NN = (((1,), (0,)), ((), ()))
_TN = (((0,), (0,)), ((), ()))


def _ln(x, g, b):
    mu = x.mean(-1, keepdims=True)
    var = ((x - mu) ** 2).mean(-1, keepdims=True)
    return (x - mu) / jnp.sqrt(var + 1e-5) * g + b


def _signed_softmax_state(s):
    s = jnp.nan_to_num(s)
    a = jnp.abs(s)
    m = jnp.max(a, axis=-1, keepdims=True)
    e = jnp.exp(a - m)
    return jnp.sign(s) * e / jnp.sum(e, axis=-1, keepdims=True) * STATE_MASS


def _topk_signed_softmax_dense(s):
    n, mw = s.shape
    c0, c1, c2, c3 = (s[:, j * 128:(j + 1) * 128] for j in range(4))
    lo01, hi01 = jnp.minimum(c0, c1), jnp.maximum(c0, c1)
    lo23, hi23 = jnp.minimum(c2, c3), jnp.maximum(c2, c3)
    r1, a02 = jnp.maximum(hi01, hi23), jnp.minimum(hi01, hi23)
    b13, r4 = jnp.maximum(lo01, lo23), jnp.minimum(lo01, lo23)
    r2, r3 = jnp.maximum(a02, b13), jnp.minimum(a02, b13)
    t = t1 = None
    for i in range(K):
        t = jnp.max(r1, axis=-1, keepdims=True)
        if i == 0:
            t1 = t
        pop = r1 == t
        r1 = jnp.where(pop, r2, r1)
        r2 = jnp.where(pop, r3, r2)
        r3 = jnp.where(pop, r4, r3)
        r4 = jnp.where(pop, -1e30, r4)
    mask = s >= t
    m1 = jnp.maximum(t1, -t)
    e = jnp.where(mask, jnp.exp(jnp.abs(s) - m1), 0.0)
    return jnp.sign(s) * e / jnp.sum(e, axis=-1, keepdims=True)


def _levels_kernel(x_ref, pos_ref, istate_ref, ival_ref, ws_ref, wd_ref, rw_ref,
                   ps_ref, pt_ref, pw_ref, g_ref, b_ref, h_ref):
    S = x_ref.shape[1]
    L, M = istate_ref.shape
    D = x_ref.shape[2]
    src = x_ref[0] + pos_ref[...]
    mem_vals = []
    mem_states = []
    for l in range(L):
        g = g_ref[l][None, :]
        bb = b_ref[l][None, :]
        state0 = _signed_softmax_state(istate_ref[l][None, :])
        val0 = _ln(ival_ref[l], g, bb)
        qs = _mm_default(src, ws_ref[l], _NN) * rw_ref[l][None, :]
        ksrc = _mm_default(val0, wd_ref[l], _NN)
        scores = _mm_default(qs, ksrc, _NT)
        w_full = _topk_signed_softmax_dense(scores)
        dval = _mm_exact(w_full, src, _TN)
        dstate = jnp.sum(w_full, axis=0, keepdims=True)
        state = _signed_softmax_state(state0 + dstate)
        val = _ln(val0 + dval, g, bb)
        pq = _mm_default(val, ps_ref[l], _NN) * pw_ref[l][None, :]
        pk = _mm_default(val, pt_ref[l], _NN)
        pscores = _mm_default(pq, pk, _NT) * state
        we_full = _topk_signed_softmax_dense(pscores)
        agg = _mm_exact(we_full, val, _NN)
        val = _ln(val + agg, g, bb)
        mem_vals.append(val)
        mem_states.append(state)
        src = val * jnp.reshape(state, (M, 1))
    memcat = jnp.concatenate(mem_vals, axis=0)
    statecat = jnp.concatenate(mem_states, axis=1)
    q = x_ref[0, S - P:, :] + pos_ref[S - P:, :]
    att_s = _mm_default(q, memcat, _NT)
    att_s = att_s / math.sqrt(D) + statecat
    att_m = jnp.max(att_s, axis=-1, keepdims=True)
    e = jnp.exp(att_s - att_m)
    att = e / jnp.sum(e, axis=-1, keepdims=True)
    read = _mm_default(att, memcat, _NN)
    h_ref[0] = q + read


def _sc_gather(emb, tok_flat):
    V, D = emb.shape
    n_tok = tok_flat.shape[0]
    info = plsc.get_sparse_core_info()
    nw = info.num_cores * info.num_subcores
    b_per_w = n_tok // nw
    mesh = plsc.VectorSubcoreMesh(core_axis_name="c", subcore_axis_name="s")

    @functools.partial(
        pl.kernel, mesh=mesh,
        out_type=jax.ShapeDtypeStruct((n_tok, D), jnp.float32),
        scratch_types=[
            pltpu.VMEM((b_per_w,), jnp.int32),
            pltpu.VMEM((b_per_w, D), jnp.float32),
            pltpu.SemaphoreType.DMA,
        ],
    )
    def k(emb_hbm, idx_hbm, out_hbm, idx_v, rows_v, sem):
        wid = lax.axis_index("s") * info.num_cores + lax.axis_index("c")
        base = wid * b_per_w
        pltpu.sync_copy(idx_hbm.at[pl.ds(base, b_per_w)], idx_v)
        pltpu.async_copy(emb_hbm.at[idx_v], rows_v, sem).wait()
        pltpu.sync_copy(rows_v, out_hbm.at[pl.ds(base, b_per_w)])

    return k(emb, tok_flat)


def _logits_kernel(h_ref, emb_ref, o_ref):
    o_ref[...] = _mm_default(h_ref[...], emb_ref[...], _NT)


def kernel(tokens, emb, pos, init_state, init_val, Ws, Wd, rw, Ps, Pt, pw, ln_g, ln_b):
    B, S = tokens.shape
    V, D = emb.shape
    L, M = init_state.shape

    x = _sc_gather(emb, tokens.reshape(-1).astype(jnp.int32))
    x = x.reshape(B, S, D)

    full = lambda a: pl.BlockSpec(a.shape, lambda bidx: (0,) * a.ndim)
    h = pl.pallas_call(
        _levels_kernel,
        grid=(B,),
        in_specs=[
            pl.BlockSpec((1, S, D), lambda bidx: (bidx, 0, 0)),
            full(pos),
            full(init_state), full(init_val), full(Ws), full(Wd), full(rw),
            full(Ps), full(Pt), full(pw), full(ln_g), full(ln_b),
        ],
        out_specs=pl.BlockSpec((1, P, D), lambda bidx: (bidx, 0, 0)),
        out_shape=jax.ShapeDtypeStruct((B, P, D), jnp.float32),
    )(x, pos, init_state, init_val, Ws, Wd, rw, Ps, Pt, pw, ln_g, ln_b)

    h2 = h.reshape(B * P, D)
    logits = pl.pallas_call(
        _logits_kernel,
        grid=(pl.cdiv(V, VB),),
        in_specs=[
            pl.BlockSpec((B * P, D), lambda i: (0, 0)),
            pl.BlockSpec((VB, D), lambda i: (i, 0)),
        ],
        out_specs=pl.BlockSpec((B * P, VB), lambda i: (0, i)),
        out_shape=jax.ShapeDtypeStruct((B * P, V), jnp.float32),
    )(h2, emb)
    return logits.reshape(B, P, V)
